# baseline (speedup 1.0000x reference)
import functools

import jax
import jax.numpy as jnp
from jax import lax
from jax.experimental import pallas as pl
from jax.experimental.pallas import tpu as pltpu
from jax.experimental.pallas import tpu_sc as plsc

EMBED = 64
NC = 2
NS = 16
NW = NC * NS
C = 256


@functools.lru_cache(maxsize=None)
def _make_lookup(Bdim, Ldim, V):
    B = Bdim * Ldim
    b_per_w = B // NW
    nchunks = b_per_w // C
    assert b_per_w % C == 0 and nchunks % 2 == 0
    assert Bdim % C == 0
    mesh = plsc.VectorSubcoreMesh(core_axis_name="c", subcore_axis_name="s")

    @functools.partial(
        pl.kernel,
        mesh=mesh,
        out_type=jax.ShapeDtypeStruct((Ldim * EMBED, Bdim), jnp.float32),
        scratch_types=[
            pltpu.VMEM((b_per_w,), jnp.int32),
            pltpu.VMEM((2, C, EMBED), jnp.float32),
            pltpu.VMEM((2, EMBED, C), jnp.float32),
            pltpu.SemaphoreType.DMA((2,)),
            pltpu.SemaphoreType.DMA((2,)),
        ],
        compiler_params=pltpu.CompilerParams(
            use_tc_tiling_on_sc=False, needs_layout_passes=False
        ),
    )
    def k(seq_hbm, tab_hbm, out_hbm, idx_v, gbuf, tbuf, gsem, wsem):
        wid = lax.axis_index("s") * NC + lax.axis_index("c")
        base = wid * b_per_w
        pltpu.sync_copy(seq_hbm.at[pl.ds(base, b_per_w)], idx_v)
        lanes = lax.iota(jnp.int32, 16)

        def fire_gather(c, bb):
            pltpu.async_copy(
                tab_hbm.at[idx_v.at[pl.ds(c * C, C)]], gbuf.at[bb], gsem.at[bb]
            )

        def wait_gather(bb):
            pltpu.make_async_copy(
                tab_hbm.at[idx_v.at[pl.ds(0, C)]], gbuf.at[bb], gsem.at[bb]
            ).wait()

        def transpose(bb):
            src = gbuf.at[bb]

            def qbody(q, carry):
                rows = q * 16 + lanes
                for e in range(EMBED):
                    cols = jnp.full((16,), e, jnp.int32)
                    vals = plsc.load_gather(src, [rows, cols])
                    tbuf[bb, e, pl.ds(q * 16, 16)] = vals
                return carry

            lax.fori_loop(0, C // 16, qbody, 0)

        def fire_write(c, bb):
            flat0 = base + c * C
            l = flat0 // Bdim
            b0 = flat0 % Bdim
            pltpu.async_copy(
                tbuf.at[bb],
                out_hbm.at[pl.ds(l * EMBED, EMBED), pl.ds(b0, C)],
                wsem.at[bb],
            )

        def wait_write(bb):
            pltpu.make_async_copy(
                tbuf.at[bb],
                out_hbm.at[pl.ds(0, EMBED), pl.ds(0, C)],
                wsem.at[bb],
            ).wait()

        fire_gather(0, 0)

        def chunk_step(c, bb):
            nxt = 1 - bb

            @pl.when(c + 1 < nchunks)
            def _():
                fire_gather(c + 1, nxt)

            wait_gather(bb)
            transpose(bb)

            @pl.when(c >= 2)
            def _():
                wait_write(bb)

            fire_write(c, bb)

        def body(g, carry):
            chunk_step(2 * g, 0)
            chunk_step(2 * g + 1, 1)
            return carry

        lax.fori_loop(0, nchunks // 2, body, 0)
        wait_write(0)
        wait_write(1)

    return k


def kernel(sequence, table):
    Bdim, Ldim = sequence.shape
    B = Bdim * Ldim
    seq_lm = sequence.T.reshape(B)
    out2d = _make_lookup(Bdim, Ldim, table.shape[0])(seq_lm, table)
    return out2d.reshape(Ldim, EMBED, Bdim).transpose(2, 0, 1)


# speedup vs baseline: 1.6908x; 1.6908x over previous
import functools

import jax
import jax.numpy as jnp
from jax import lax
from jax.experimental import pallas as pl
from jax.experimental.pallas import tpu as pltpu
from jax.experimental.pallas import tpu_sc as plsc

EMBED = 64
NC = 2
NS = 16
NW = NC * NS

NBUF = 4
AHEAD = 2


@functools.lru_cache(maxsize=None)
def _make_gather(B, C):
    b_per_w = B // NW
    nchunks = b_per_w // C
    assert b_per_w % C == 0
    assert (nchunks - 2 * AHEAD) % NBUF == 0 and nchunks >= 2 * AHEAD + NBUF
    mesh = plsc.VectorSubcoreMesh(core_axis_name="c", subcore_axis_name="s")

    @functools.partial(
        pl.kernel,
        mesh=mesh,
        out_type=jax.ShapeDtypeStruct((B, EMBED), jnp.float32),
        scratch_types=[
            pltpu.VMEM((b_per_w,), jnp.int32),
            pltpu.VMEM((NBUF, C, EMBED), jnp.float32),
            pltpu.SemaphoreType.DMA((NBUF,)),
            pltpu.SemaphoreType.DMA((NBUF,)),
        ],
        compiler_params=pltpu.CompilerParams(use_tc_tiling_on_sc=False),
    )
    def k(seq_hbm, table_hbm, out_hbm, idx_v, bufs, gsem, wsem):
        wid = lax.axis_index("s") * NC + lax.axis_index("c")
        base = wid * b_per_w
        pltpu.sync_copy(seq_hbm.at[pl.ds(base, b_per_w)], idx_v)

        def fire_gather(c, b):
            pltpu.async_copy(
                table_hbm.at[idx_v.at[pl.ds(c * C, C)]], bufs.at[b], gsem.at[b]
            )

        def wait_gather(b):
            pltpu.make_async_copy(
                table_hbm.at[idx_v.at[pl.ds(0, C)]], bufs.at[b], gsem.at[b]
            ).wait()

        def fire_write(c, b):
            pltpu.async_copy(
                bufs.at[b], out_hbm.at[pl.ds(base + c * C, C)], wsem.at[b]
            )

        def wait_write(b):
            pltpu.make_async_copy(
                bufs.at[b], out_hbm.at[pl.ds(base, C)], wsem.at[b]
            ).wait()

        for c in range(AHEAD):
            fire_gather(c, c % NBUF)
        for c in range(AHEAD):
            b = c % NBUF
            wait_gather(b)
            fire_write(c, b)
            fire_gather(c + AHEAD, (c + AHEAD) % NBUF)

        ngroups = (nchunks - 2 * AHEAD) // NBUF

        def body(g, carry):
            c0 = AHEAD + NBUF * g
            for j in range(NBUF):
                c = c0 + j
                b = (AHEAD + j) % NBUF
                wait_gather(b)
                fire_write(c, b)
                b2 = (AHEAD + j + AHEAD) % NBUF
                wait_write(b2)
                fire_gather(c + AHEAD, b2)
            return carry

        lax.fori_loop(0, ngroups, body, 0)

        for c in range(nchunks - AHEAD, nchunks):
            b = c % NBUF
            wait_gather(b)
            fire_write(c, b)
        for c in range(nchunks - NBUF, nchunks):
            wait_write(c % NBUF)

    return k


def kernel(sequence, table):
    Bdim, Ldim = sequence.shape
    B = Bdim * Ldim
    seq_lm = sequence.T.reshape(B)
    out = _make_gather(B, 320)(seq_lm, table)
    return out.reshape(Ldim, Bdim, EMBED).transpose(1, 0, 2)
